# baseline (speedup 1.0000x reference)
import functools

import jax
import jax.numpy as jnp
from jax import lax
from jax.experimental import pallas as pl
from jax.experimental.pallas import tpu as pltpu
from jax.experimental.pallas import tpu_sc as plsc

_VOCAB = 100000
_EMB = 128
_BATCH = 4096
_FIELDS = 100
_NC = 2
_NS = 16
_NW = _NC * _NS
_BPW = _BATCH // _NW
_GRP = _BPW // 16
_VCHUNK = 4096
_VGRID = -(-_VOCAB // _VCHUNK)
_CCH = 10


def _c_table_body(t2_ref, t1r_ref, c_ref):
    t2 = t2_ref[...]
    ones = jnp.ones((1, _EMB), jnp.float32)
    norm2 = jax.lax.dot_general(
        ones, t2 * t2, (((1,), (1,)), ((), ())),
        preferred_element_type=jnp.float32,
    )
    c_ref[...] = t1r_ref[...] - 0.5 * norm2


def _c_table(order2_table, order1_row):
    return pl.pallas_call(
        _c_table_body,
        grid=(_VGRID,),
        in_specs=[
            pl.BlockSpec((_VCHUNK, _EMB), lambda i: (i, 0)),
            pl.BlockSpec((1, _VCHUNK), lambda i: (0, i)),
        ],
        out_specs=pl.BlockSpec((1, _VCHUNK), lambda i: (0, i)),
        out_shape=jax.ShapeDtypeStruct((1, _VOCAB), jnp.float32),
    )(order2_table, order1_row)


_mesh = plsc.VectorSubcoreMesh(
    core_axis_name="c", subcore_axis_name="s", num_cores=_NC, num_subcores=_NS
)


@functools.partial(
    pl.kernel,
    out_type=jax.ShapeDtypeStruct((_BATCH,), jnp.float32),
    mesh=_mesh,
    scratch_types=[
        pltpu.VMEM((_BPW, _FIELDS), jnp.int32),
        pltpu.VMEM((_FIELDS, _BPW), jnp.int32),
        pltpu.VMEM((_BPW, _EMB), jnp.float32),
        pltpu.VMEM((_BPW,), jnp.float32),
        pltpu.SemaphoreType.DMA,
        pltpu.SemaphoreType.DMA,
    ],
    compiler_params=pltpu.CompilerParams(needs_layout_passes=False),
)
def _fm_sc_a(idx_hbm, t2_hbm, out_hbm, idx_v, frows_v, acc_v, out_v, isem, gsem):
    _zeros16 = jnp.zeros((16,), jnp.float32)
    cid = lax.axis_index("c")
    sid = lax.axis_index("s")
    wid = sid * _NC + cid
    base = wid * _BPW

    idx_cp = pltpu.make_async_copy(idx_hbm.at[pl.ds(base, _BPW), :], idx_v, isem)
    idx_cp.start()

    rows = [jnp.arange(16, dtype=jnp.int32) + g * 16 for g in range(_GRP)]

    def _zero(b, carry):
        for j in range(_EMB // 16):
            acc_v[b, pl.ds(j * 16, 16)] = _zeros16
        return carry

    lax.fori_loop(0, _BPW, _zero, 0)
    idx_cp.wait()

    def _field(f, carry):
        col = jnp.full((16,), f, jnp.int32)
        for g in range(_GRP):
            frows_v[f, pl.ds(g * 16, 16)] = plsc.load_gather(idx_v, [rows[g], col])
        pltpu.async_copy(t2_hbm.at[frows_v.at[f]], acc_v, gsem, add=True)
        return carry

    lax.fori_loop(0, _FIELDS, _field, 0)

    def _drain(f, carry):
        pltpu.make_async_copy(t2_hbm.at[frows_v.at[0]], acc_v, gsem).wait()
        return carry

    lax.fori_loop(0, _FIELDS, _drain, 0)

    def _ssq(d, ssq):
        col = jnp.full((16,), d, jnp.int32)
        out = []
        for g in range(_GRP):
            v = plsc.load_gather(acc_v, [rows[g], col])
            out.append(ssq[g] + v * v)
        return tuple(out)

    ssq = lax.fori_loop(0, _EMB, _ssq, (_zeros16,) * _GRP)

    for g in range(_GRP):
        out_v[pl.ds(g * 16, 16)] = 0.5 * ssq[g]

    pltpu.sync_copy(out_v, out_hbm.at[pl.ds(base, _BPW)])


@functools.partial(
    pl.kernel,
    out_type=jax.ShapeDtypeStruct((_BATCH,), jnp.float32),
    mesh=_mesh,
    scratch_types=[
        pltpu.VMEM((_BPW, _FIELDS), jnp.int32),
        pltpu.VMEM((_VOCAB,), jnp.float32),
        pltpu.VMEM((_BPW,), jnp.float32),
        pltpu.SemaphoreType.DMA,
        pltpu.SemaphoreType.DMA,
        pltpu.SemaphoreType.DMA,
    ],
    compiler_params=pltpu.CompilerParams(needs_layout_passes=False),
)
def _fm_sc_b(idx_hbm, c_hbm, part_hbm, out_hbm, idx_v, c_v, out_v,
             csem, isem, psem):
    _zeros16 = jnp.zeros((16,), jnp.float32)
    cid = lax.axis_index("c")
    sid = lax.axis_index("s")
    wid = sid * _NC + cid
    base = wid * _BPW
    cw = _VOCAB // _CCH

    idx_cp = pltpu.make_async_copy(idx_hbm.at[pl.ds(base, _BPW), :], idx_v, isem)
    idx_cp.start()
    part_cp = pltpu.make_async_copy(part_hbm.at[pl.ds(base, _BPW)], out_v, psem)
    part_cp.start()
    for k in range(_CCH):
        chunk = (wid + k) % _CCH
        pltpu.make_async_copy(
            c_hbm.at[pl.ds(chunk * cw, cw)], c_v.at[pl.ds(chunk * cw, cw)], csem
        ).start()
    idx_cp.wait()
    part_cp.wait()

    rows = [jnp.arange(16, dtype=jnp.int32) + g * 16 for g in range(_GRP)]

    for k in range(_CCH):
        pltpu.make_async_copy(
            c_hbm.at[pl.ds(0, cw)], c_v.at[pl.ds(0, cw)], csem
        ).wait()

    col0 = jnp.full((16,), 0, jnp.int32)
    i16_0 = tuple(plsc.load_gather(idx_v, [rows[g], col0]) for g in range(_GRP))

    def _csum(f, carry):
        cacc, i16 = carry
        coln = jnp.full((16,), f + 1, jnp.int32)
        nxt, out = [], []
        for g in range(_GRP):
            out.append(cacc[g] + plsc.load_gather(c_v, [i16[g]]))
            nxt.append(plsc.load_gather(idx_v, [rows[g], coln]))
        return tuple(out), tuple(nxt)

    cacc, i16_l = lax.fori_loop(0, _FIELDS - 1, _csum, ((_zeros16,) * _GRP, i16_0))
    cacc = tuple(cacc[g] + plsc.load_gather(c_v, [i16_l[g]]) for g in range(_GRP))

    for g in range(_GRP):
        out_v[pl.ds(g * 16, 16)] = out_v[pl.ds(g * 16, 16)] + cacc[g]

    pltpu.sync_copy(out_v, out_hbm.at[pl.ds(base, _BPW)])


def kernel(inputs, order2_table, order1_table):
    idx = inputs.astype(jnp.int32)
    t1_row = order1_table.reshape(1, _VOCAB)
    c = _c_table(order2_table, t1_row).reshape(_VOCAB)
    partial = _fm_sc_a(idx, order2_table)
    out = _fm_sc_b(idx, c, partial)
    return out.reshape(_BATCH, 1)


# speedup vs baseline: 1.0204x; 1.0204x over previous
import functools

import jax
import jax.numpy as jnp
from jax import lax
from jax.experimental import pallas as pl
from jax.experimental.pallas import tpu as pltpu
from jax.experimental.pallas import tpu_sc as plsc

_VOCAB = 100000
_EMB = 128
_BATCH = 4096
_FIELDS = 100
_NC = 2
_NS = 16
_NW = _NC * _NS
_BPW = _BATCH // _NW
_GRP = _BPW // 16
_VCHUNK = 4096
_VGRID = -(-_VOCAB // _VCHUNK)
_CCH = 10


def _c_table_body(t2_ref, t1r_ref, c_ref):
    t2 = t2_ref[...]
    ones = jnp.ones((1, _EMB), jnp.float32)
    norm2 = jax.lax.dot_general(
        ones, t2 * t2, (((1,), (1,)), ((), ())),
        preferred_element_type=jnp.float32,
    )
    c_ref[...] = t1r_ref[...] - 0.5 * norm2


def _c_table(order2_table, order1_row):
    return pl.pallas_call(
        _c_table_body,
        grid=(_VGRID,),
        in_specs=[
            pl.BlockSpec((_VCHUNK, _EMB), lambda i: (i, 0)),
            pl.BlockSpec((1, _VCHUNK), lambda i: (0, i)),
        ],
        out_specs=pl.BlockSpec((1, _VCHUNK), lambda i: (0, i)),
        out_shape=jax.ShapeDtypeStruct((1, _VOCAB), jnp.float32),
    )(order2_table, order1_row)


_mesh = plsc.VectorSubcoreMesh(
    core_axis_name="c", subcore_axis_name="s", num_cores=_NC, num_subcores=_NS
)


@functools.partial(
    pl.kernel,
    out_type=jax.ShapeDtypeStruct((_BATCH,), jnp.float32),
    mesh=_mesh,
    scratch_types=[
        pltpu.VMEM((_BPW, _FIELDS), jnp.int32),
        pltpu.VMEM((_FIELDS, _BPW), jnp.int32),
        pltpu.VMEM((_BPW, _EMB), jnp.float32),
        pltpu.VMEM((_BPW,), jnp.float32),
        pltpu.SemaphoreType.DMA,
        pltpu.SemaphoreType.DMA,
    ],
    compiler_params=pltpu.CompilerParams(needs_layout_passes=False),
)
def _fm_sc_a(idx_hbm, t2_hbm, out_hbm, idx_v, frows_v, acc_v, out_v, isem, gsem):
    _zeros16 = jnp.zeros((16,), jnp.float32)
    cid = lax.axis_index("c")
    sid = lax.axis_index("s")
    wid = sid * _NC + cid
    base = wid * _BPW

    idx_cp = pltpu.make_async_copy(idx_hbm.at[pl.ds(base, _BPW), :], idx_v, isem)
    idx_cp.start()

    rows = [jnp.arange(16, dtype=jnp.int32) + g * 16 for g in range(_GRP)]

    def _zero(b, carry):
        for j in range(_EMB // 16):
            acc_v[b, pl.ds(j * 16, 16)] = _zeros16
        return carry

    lax.fori_loop(0, _BPW, _zero, 0)
    idx_cp.wait()

    def _field(f, carry):
        col = jnp.full((16,), f, jnp.int32)
        for g in range(_GRP):
            frows_v[f, pl.ds(g * 16, 16)] = plsc.load_gather(idx_v, [rows[g], col])
        pltpu.async_copy(t2_hbm.at[frows_v.at[f]], acc_v, gsem, add=True)
        return carry

    lax.fori_loop(0, _FIELDS, _field, 0)

    def _drain(f, carry):
        pltpu.make_async_copy(t2_hbm.at[frows_v.at[0]], acc_v, gsem).wait()
        return carry

    lax.fori_loop(0, _FIELDS, _drain, 0)

    def _ssq(d, ssq):
        col = jnp.full((16,), d, jnp.int32)
        out = []
        for g in range(_GRP):
            v = plsc.load_gather(acc_v, [rows[g], col])
            out.append(ssq[g] + v * v)
        return tuple(out)

    ssq = lax.fori_loop(0, _EMB, _ssq, (_zeros16,) * _GRP)

    for g in range(_GRP):
        out_v[pl.ds(g * 16, 16)] = 0.5 * ssq[g]

    pltpu.sync_copy(out_v, out_hbm.at[pl.ds(base, _BPW)])


@functools.partial(
    pl.kernel,
    out_type=jax.ShapeDtypeStruct((_BATCH,), jnp.float32),
    mesh=_mesh,
    scratch_types=[
        pltpu.VMEM((_BPW, _FIELDS), jnp.int32),
        pltpu.VMEM((_FIELDS, _BPW), jnp.int32),
        pltpu.VMEM((_VOCAB,), jnp.float32),
        pltpu.VMEM((_BPW,), jnp.float32),
        pltpu.SemaphoreType.DMA,
        pltpu.SemaphoreType.DMA,
        pltpu.SemaphoreType.DMA,
    ],
    compiler_params=pltpu.CompilerParams(needs_layout_passes=False),
)
def _fm_sc_b(idx_hbm, c_hbm, part_hbm, out_hbm, idx_v, frows_v, c_v, out_v,
             csem, isem, psem):
    _zeros16 = jnp.zeros((16,), jnp.float32)
    cid = lax.axis_index("c")
    sid = lax.axis_index("s")
    wid = sid * _NC + cid
    base = wid * _BPW
    cw = _VOCAB // _CCH

    idx_cp = pltpu.make_async_copy(idx_hbm.at[pl.ds(base, _BPW), :], idx_v, isem)
    idx_cp.start()
    part_cp = pltpu.make_async_copy(part_hbm.at[pl.ds(base, _BPW)], out_v, psem)
    part_cp.start()
    for k in range(_CCH):
        chunk = (wid + k) % _CCH
        pltpu.make_async_copy(
            c_hbm.at[pl.ds(chunk * cw, cw)], c_v.at[pl.ds(chunk * cw, cw)], csem
        ).start()
    idx_cp.wait()
    part_cp.wait()

    rows = [jnp.arange(16, dtype=jnp.int32) + g * 16 for g in range(_GRP)]

    def _build(f, carry):
        col = jnp.full((16,), f, jnp.int32)
        for g in range(_GRP):
            frows_v[f, pl.ds(g * 16, 16)] = plsc.load_gather(idx_v, [rows[g], col])
        return carry

    lax.fori_loop(0, _FIELDS, _build, 0)

    for k in range(_CCH):
        pltpu.make_async_copy(
            c_hbm.at[pl.ds(0, cw)], c_v.at[pl.ds(0, cw)], csem
        ).wait()

    def _csum(f, cacc):
        out = []
        for g in range(_GRP):
            i16 = frows_v[f, pl.ds(g * 16, 16)]
            out.append(cacc[g] + plsc.load_gather(c_v, [i16]))
        return tuple(out)

    cacc = lax.fori_loop(0, _FIELDS, _csum, (_zeros16,) * _GRP)

    for g in range(_GRP):
        out_v[pl.ds(g * 16, 16)] = out_v[pl.ds(g * 16, 16)] + cacc[g]

    pltpu.sync_copy(out_v, out_hbm.at[pl.ds(base, _BPW)])


def kernel(inputs, order2_table, order1_table):
    idx = inputs.astype(jnp.int32)
    t1_row = order1_table.reshape(1, _VOCAB)
    c = _c_table(order2_table, t1_row).reshape(_VOCAB)
    partial = _fm_sc_a(idx, order2_table)
    out = _fm_sc_b(idx, c, partial)
    return out.reshape(_BATCH, 1)
